# baseline (speedup 1.0000x reference)
import functools

import jax
import jax.numpy as jnp
from jax import lax
from jax.experimental import pallas as pl
from jax.experimental.pallas import tpu as pltpu
from jax.experimental.pallas import tpu_sc as plsc

N = 10000
NPAD = 10048
ROWS_PER_TILE = NPAD // 16
E = 320000
CH = 64
EROWS = 5120
M_A = 40
M_C = 20
D = 128
DO = 64

_mesh = plsc.VectorSubcoreMesh(core_axis_name="c", subcore_axis_name="s")


def _even_odd_perm(width):
    p = []
    for k in range(width // 32):
        p += [32 * k + 2 * i for i in range(16)]
        p += [32 * k + 2 * i + 1 for i in range(16)]
    return p


_PERM128 = _even_odd_perm(D)
_P64 = _even_odd_perm(DO)
_INV64 = [0] * DO
for _pos, _e in enumerate(_P64):
    _INV64[_e] = _pos


def _bf16_to_f32(brow, frow, width):
    def rowfn(r, c):
        for k in range(width // 32):
            v = plsc.bitcast(brow[r, pl.ds(32 * k, 32)], jnp.int32)
            lo = plsc.bitcast(lax.shift_left(v, 16), jnp.float32)
            hi = plsc.bitcast(lax.bitwise_and(v, jnp.int32(-65536)),
                              jnp.float32)
            frow[r, pl.ds(32 * k, 16)] = lo
            frow[r, pl.ds(32 * k + 16, 16)] = hi
        return c

    lax.fori_loop(0, CH, rowfn, 0)


def _edge_pipeline(tab, s2d, d2d, row0, niter, width,
                   idx_s, idx_d, brows, frows, sem_g, sem_s, sem_i,
                   acc_sh, onesv, cnt_sh):
    def load_idx(k, q):
        r = row0 + k * 2
        pltpu.async_copy(s2d.at[pl.ds(r, 2)], idx_s[q], sem_i[q])
        pltpu.async_copy(d2d.at[pl.ds(r, 2)], idx_d[q], sem_i[q])

    def wait_idx(k, q):
        r = row0 + k * 2
        pltpu.make_async_copy(s2d.at[pl.ds(r, 2)], idx_s[q],
                              sem_i[q]).wait()
        pltpu.make_async_copy(d2d.at[pl.ds(r, 2)], idx_d[q],
                              sem_i[q]).wait()

    def issue_gathers(q):
        p = 2 * (q % 2)
        for j in range(2):
            pltpu.async_copy(tab.at[idx_s[q].at[j]], brows[p + j],
                             sem_g[p + j])

    def wait_gathers(q):
        p = 2 * (q % 2)
        for j in range(2):
            pltpu.make_async_copy(tab.at[idx_s[q].at[j]], brows[p + j],
                                  sem_g[p + j]).wait()

    def convert(q):
        p = 2 * (q % 2)
        for j in range(2):
            _bf16_to_f32(brows[p + j], frows[j], width)

    def issue_scatters(q):
        for j in range(2):
            pltpu.async_copy(frows[j], acc_sh.at[idx_d[q].at[j]], sem_s[j],
                             add=True)
            if cnt_sh is not None:
                pltpu.async_copy(onesv, cnt_sh.at[idx_d[q].at[j]], sem_s[j],
                                 add=True)

    def wait_scatters(q):
        for j in range(2):
            pltpu.make_async_copy(frows[j], acc_sh.at[idx_d[q].at[j]],
                                  sem_s[j]).wait()
            if cnt_sh is not None:
                pltpu.make_async_copy(onesv, cnt_sh.at[idx_d[q].at[j]],
                                      sem_s[j]).wait()

    load_idx(0, 0)
    load_idx(1, 1)
    load_idx(2, 2)
    wait_idx(0, 0)
    issue_gathers(0)
    wait_idx(1, 1)
    issue_gathers(1)

    def body(m, carry):
        g0 = 4 * m
        for q in range(4):
            wait_gathers(q)

            if q == 0:
                @pl.when(m > 0)
                def _():
                    wait_scatters(3)
            else:
                wait_scatters(q - 1)

            convert(q)
            issue_scatters(q)

            if q == 0:
                load_idx(g0 + 3, 3)
                wait_idx(g0 + 2, 2)
                issue_gathers(2)
            elif q == 1:
                @pl.when(m + 1 < niter)
                def _():
                    load_idx(g0 + 4, 0)
                wait_idx(g0 + 3, 3)
                issue_gathers(3)
            else:
                @pl.when(m + 1 < niter)
                def _(q=q, g0=g0):
                    load_idx(g0 + q + 3, (q + 3) % 4)
                    wait_idx(g0 + q + 2, (q + 2) % 4)
                    issue_gathers((q + 2) % 4)

        return carry

    lax.fori_loop(0, niter, body, 0)

    wait_scatters(3)


_TILE_CHUNKS = [(t * CH, CH) for t in range(ROWS_PER_TILE // CH)]
if ROWS_PER_TILE % CH:
    _TILE_CHUNKS.append((ROWS_PER_TILE - ROWS_PER_TILE % CH,
                         ROWS_PER_TILE % CH))


def _stage_a_body(xc, xp, s_all, d_all, z128, z16, ones_h,
                  aggp, cntp, aggc, cntc,
                  i_s0, i_s1, i_s2, i_s3, i_d0, i_d1, i_d2, i_d3,
                  b0, b1, b2, b3, f0, f1,
                  onesv, acc_sh, cnt_sh,
                  sg0, sg1, sg2, sg3, ss0, ss1,
                  si0, si1, si2, si3):
    idx_s = [i_s0, i_s1, i_s2, i_s3]
    idx_d = [i_d0, i_d1, i_d2, i_d3]
    brows = [b0, b1, b2, b3]
    frows = [f0, f1]
    sem_g = [sg0, sg1, sg2, sg3]
    sem_s = [ss0, ss1]
    sem_i = [si0, si1, si2, si3]

    cid = lax.axis_index("c")
    sid = lax.axis_index("s")
    row0 = sid * ROWS_PER_TILE

    pltpu.sync_copy(z16, onesv)
    pltpu.sync_copy(z128, f0)
    for off, nr in _TILE_CHUNKS:
        pltpu.sync_copy(f0.at[pl.ds(0, nr)],
                        acc_sh.at[pl.ds(row0 + off, nr)])
        pltpu.sync_copy(onesv.at[pl.ds(0, nr)],
                        cnt_sh.at[pl.ds(row0 + off, nr)])
    pltpu.sync_copy(ones_h, onesv)
    plsc.subcore_barrier()

    erow0 = sid * (M_A * 8)

    @pl.when(cid == 0)
    def _():
        _edge_pipeline(xc, s_all.at[0], d_all.at[0], erow0, M_A, D,
                       idx_s, idx_d, brows, frows, sem_g, sem_s, sem_i,
                       acc_sh, onesv, cnt_sh)

    @pl.when(cid == 1)
    def _():
        _edge_pipeline(xp, s_all.at[1], d_all.at[1], erow0, M_A, D,
                       idx_s, idx_d, brows, frows, sem_g, sem_s, sem_i,
                       acc_sh, onesv, cnt_sh)

    plsc.subcore_barrier()

    def _writeback(agg_out, cnt_out):
        for off, nr in _TILE_CHUNKS:
            r = row0 + off
            pltpu.sync_copy(acc_sh.at[pl.ds(r, nr)], f0.at[pl.ds(0, nr)])
            pltpu.sync_copy(f0.at[pl.ds(0, nr)], agg_out.at[pl.ds(r, nr)])
            pltpu.sync_copy(cnt_sh.at[pl.ds(r, nr)], onesv.at[pl.ds(0, nr)])
            pltpu.sync_copy(onesv.at[pl.ds(0, nr)], cnt_out.at[pl.ds(r, nr)])

    @pl.when(cid == 0)
    def _():
        _writeback(aggp, cntp)

    @pl.when(cid == 1)
    def _():
        _writeback(aggc, cntc)


_stage_a = functools.partial(
    pl.kernel,
    out_type=[
        jax.ShapeDtypeStruct((NPAD, D), jnp.float32),
        jax.ShapeDtypeStruct((NPAD, 16), jnp.float32),
        jax.ShapeDtypeStruct((NPAD, D), jnp.float32),
        jax.ShapeDtypeStruct((NPAD, 16), jnp.float32),
    ],
    mesh=_mesh,
    scratch_types=(
        [pltpu.VMEM((2, CH), jnp.int32)] * 8 +
        [pltpu.VMEM((CH, D), jnp.bfloat16)] * 4 +
        [pltpu.VMEM((CH, D), jnp.float32)] * 2 +
        [pltpu.VMEM((CH, 16), jnp.float32)] +
        [pltpu.VMEM_SHARED((NPAD, D), jnp.float32),
         pltpu.VMEM_SHARED((NPAD, 16), jnp.float32)]
        + [pltpu.SemaphoreType.DMA] * 10
    ),
    compiler_params=pltpu.CompilerParams(use_tc_tiling_on_sc=False,
                                         needs_layout_passes=False),
)(_stage_a_body)


def _stage_c_body(g, s_all, d_all, z64,
                  agg2,
                  i_s0, i_s1, i_s2, i_s3, i_d0, i_d1, i_d2, i_d3,
                  b0, b1, b2, b3, f0, f1,
                  acc_sh,
                  sg0, sg1, sg2, sg3, ss0, ss1,
                  si0, si1, si2, si3):
    idx_s = [i_s0, i_s1, i_s2, i_s3]
    idx_d = [i_d0, i_d1, i_d2, i_d3]
    brows = [b0, b1, b2, b3]
    frows = [f0, f1]
    sem_g = [sg0, sg1, sg2, sg3]
    sem_s = [ss0, ss1]
    sem_i = [si0, si1, si2, si3]

    cid = lax.axis_index("c")
    sid = lax.axis_index("s")
    row0 = sid * ROWS_PER_TILE

    pltpu.sync_copy(z64, f0)
    for off, nr in _TILE_CHUNKS:
        pltpu.sync_copy(f0.at[pl.ds(0, nr)],
                        acc_sh.at[pl.ds(row0 + off, nr)])
    plsc.subcore_barrier()

    wid = sid * 2 + cid
    erow0 = wid * (M_C * 8)
    _edge_pipeline(g, s_all.at[1], d_all.at[1], erow0, M_C, DO,
                   idx_s, idx_d, brows, frows, sem_g, sem_s, sem_i,
                   acc_sh, None, None)

    plsc.subcore_barrier()

    def _writeback(out2d):
        for off, nr in _TILE_CHUNKS:
            r = row0 + off
            pltpu.sync_copy(acc_sh.at[pl.ds(r, nr)], f0.at[pl.ds(0, nr)])
            pltpu.sync_copy(f0.at[pl.ds(0, nr)], out2d.at[pl.ds(r, nr)])

    @pl.when(cid == 0)
    def _():
        _writeback(agg2.at[0])

    @pl.when(cid == 1)
    def _():
        _writeback(agg2.at[1])


_stage_c = functools.partial(
    pl.kernel,
    out_type=[jax.ShapeDtypeStruct((2, NPAD, DO), jnp.float32)],
    mesh=_mesh,
    scratch_types=(
        [pltpu.VMEM((2, CH), jnp.int32)] * 8 +
        [pltpu.VMEM((CH, DO), jnp.bfloat16)] * 4 +
        [pltpu.VMEM((CH, DO), jnp.float32)] * 2 +
        [pltpu.VMEM_SHARED((NPAD, DO), jnp.float32)]
        + [pltpu.SemaphoreType.DMA] * 10
    ),
    compiler_params=pltpu.CompilerParams(use_tc_tiling_on_sc=False,
                                         needs_layout_passes=False),
)(_stage_c_body)


_BLK = 1000


def _dot(a, b):
    return jnp.dot(a, b, preferred_element_type=jnp.float32,
                   precision=lax.Precision.HIGHEST)


def _stage_b_kern(aggp, cntp, xp, aggc, cntc, xc,
                  W1cl, b1c, W1cr, W1pl, b1p, W1pr,
                  W2pl, W2pr, WL, WLg, b2p, bL,
                  g_out, z_out):
    mean_p = aggp[...] / jnp.maximum(cntp[:, 0:1], 1.0)
    h_p = jnp.maximum(
        _dot(mean_p, W1cl[...]) + b1c[...] + _dot(xp[...], W1cr[...]), 0.0)
    g_out[...] = _dot(h_p, _dot(W2pl[...], WLg[...])).astype(jnp.bfloat16)

    mean_c = aggc[...] / jnp.maximum(cntc[:, 0:1], 1.0)
    h_c = jnp.maximum(
        _dot(mean_c, W1pl[...]) + b1p[...] + _dot(xc[...], W1pr[...]), 0.0)
    z_out[...] = (_dot(h_c, _dot(W2pr[...], WL[...]))
                  + _dot(b2p[...], WL[...]) + bL[...])


def _stage_b(aggp, cntp, xp, aggc, cntc, xc,
             W1cl, b1c, W1cr, W1pl, b1p, W1pr, W2pl, W2pr, WL, WLg,
             b2p, bL):
    row_spec = lambda w: pl.BlockSpec((_BLK, w), lambda i: (i, 0))
    full = lambda a: pl.BlockSpec(a.shape, lambda i: (0,) * a.ndim)
    return pl.pallas_call(
        _stage_b_kern,
        grid=(N // _BLK,),
        in_specs=[
            row_spec(D), row_spec(16), row_spec(D),
            row_spec(D), row_spec(16), row_spec(D),
            full(W1cl), full(b1c), full(W1cr),
            full(W1pl), full(b1p), full(W1pr),
            full(W2pl), full(W2pr), full(WL), full(WLg), full(b2p),
            full(bL),
        ],
        out_specs=[row_spec(DO), row_spec(DO)],
        out_shape=[
            jax.ShapeDtypeStruct((N, DO), jnp.bfloat16),
            jax.ShapeDtypeStruct((N, DO), jnp.float32),
        ],
    )(aggp, cntp, xp, aggc, cntc, xc,
      W1cl, b1c, W1cr, W1pl, b1p, W1pr, W2pl, W2pr, WL, WLg, b2p, bL)


def _stage_d_kern(p0, p1, cntc, z, out):
    out[...] = ((p0[...] + p1[...]) / jnp.maximum(cntc[:, 0:1], 1.0)
                + z[...])


def _stage_d(p0, p1, cntc, z):
    row_spec = lambda w: pl.BlockSpec((_BLK, w), lambda i: (i, 0))
    return pl.pallas_call(
        _stage_d_kern,
        grid=(N // _BLK,),
        in_specs=[row_spec(DO), row_spec(DO), row_spec(16), row_spec(DO)],
        out_specs=row_spec(DO),
        out_shape=jax.ShapeDtypeStruct((N, DO), jnp.float32),
    )(p0, p1, cntc, z)


def _pad_edges(ei):
    src = ei[0].astype(jnp.int32)
    dst = ei[1].astype(jnp.int32)
    pad = EROWS * CH - E
    src = jnp.concatenate([src, jnp.zeros((pad,), jnp.int32)])
    dst = jnp.concatenate([dst, jnp.full((pad,), N, jnp.int32)])
    return src.reshape(EROWS, CH), dst.reshape(EROWS, CH)


def kernel(x_customer, x_product, edge_index_cbp, edge_index_pbc,
           W1_cbp_l, b1_cbp, W1_cbp_r, W1_pbc_l, b1_pbc, W1_pbc_r,
           W2_cbp_l, b2_cbp, W2_cbp_r, W2_pbc_l, b2_pbc, W2_pbc_r,
           W_lin, b_lin):
    s_cbp, d_cbp = _pad_edges(edge_index_cbp)
    s_pbc, d_pbc = _pad_edges(edge_index_pbc)
    s_all = jnp.stack([s_cbp, s_pbc])
    d_all = jnp.stack([d_cbp, d_pbc])

    z128 = jnp.zeros((CH, D), jnp.float32)
    z16 = jnp.zeros((CH, 16), jnp.float32)
    z64 = jnp.zeros((CH, DO), jnp.float32)
    ones_h = jnp.ones((CH, 16), jnp.float32)

    xc_bf = x_customer.astype(jnp.bfloat16)
    xp_bf = x_product.astype(jnp.bfloat16)

    aggp, cntp, aggc, cntc = _stage_a(
        xc_bf, xp_bf, s_all, d_all, z128, z16, ones_h)

    perm = jnp.array(_PERM128, jnp.int32)
    g, z = _stage_b(
        aggp[:N], cntp[:N], x_product, aggc[:N], cntc[:N], x_customer,
        W1_cbp_l[perm], b1_cbp.reshape(1, D), W1_cbp_r,
        W1_pbc_l[perm], b1_pbc.reshape(1, D), W1_pbc_r,
        W2_pbc_l, W2_pbc_r, W_lin, W_lin[:, jnp.array(_INV64, jnp.int32)],
        b2_pbc.reshape(1, D), b_lin.reshape(1, DO))

    (agg2,) = _stage_c(g, s_all, d_all, z64)

    return _stage_d(agg2[0, :N], agg2[1, :N], cntc[:N], z)


# speedup vs baseline: 1.0109x; 1.0109x over previous
import functools

import jax
import jax.numpy as jnp
from jax import lax
from jax.experimental import pallas as pl
from jax.experimental.pallas import tpu as pltpu
from jax.experimental.pallas import tpu_sc as plsc

N = 10000
NPAD = 10048
ROWS_PER_TILE = NPAD // 16
E = 320000
CH = 128
NCHUNK = 2560
M_A = 40
M_C = 20
D = 128
DO = 64

_mesh = plsc.VectorSubcoreMesh(core_axis_name="c", subcore_axis_name="s")


def _even_odd_perm(width):
    p = []
    for k in range(width // 32):
        p += [32 * k + 2 * i for i in range(16)]
        p += [32 * k + 2 * i + 1 for i in range(16)]
    return p


_PERM128 = _even_odd_perm(D)
_P64 = _even_odd_perm(DO)
_INV64 = [0] * DO
for _pos, _e in enumerate(_P64):
    _INV64[_e] = _pos


def _bf16_to_f32(brow, frow, width):
    def rowfn(r, c):
        for k in range(width // 32):
            v = plsc.bitcast(brow[r, pl.ds(32 * k, 32)], jnp.int32)
            lo = plsc.bitcast(lax.shift_left(v, 16), jnp.float32)
            hi = plsc.bitcast(lax.bitwise_and(v, jnp.int32(-65536)),
                              jnp.float32)
            frow[r, pl.ds(32 * k, 16)] = lo
            frow[r, pl.ds(32 * k + 16, 16)] = hi
        return c

    lax.fori_loop(0, CH, rowfn, 0)


def _edge_pipeline(tab, sd2d, chunk0, niter, width,
                   idx, brows, frows, sem_g, sem_s, sem_i,
                   acc_sh, onesv, cnt_sh):
    def load_idx(k, q):
        pltpu.async_copy(sd2d.at[chunk0 + k], idx[q], sem_i[q])

    def wait_idx(k, q):
        pltpu.make_async_copy(sd2d.at[chunk0 + k], idx[q], sem_i[q]).wait()

    def issue_gather(q):
        b = q % 2
        pltpu.async_copy(tab.at[idx[q].at[0]], brows[b], sem_g[b])

    def wait_gather(q):
        b = q % 2
        pltpu.make_async_copy(tab.at[idx[q].at[0]], brows[b],
                              sem_g[b]).wait()

    def issue_scatter(q):
        pltpu.async_copy(frows, acc_sh.at[idx[q].at[1]], sem_s, add=True)
        if cnt_sh is not None:
            pltpu.async_copy(onesv, cnt_sh.at[idx[q].at[1]], sem_s,
                             add=True)

    def wait_scatter(q):
        pltpu.make_async_copy(frows, acc_sh.at[idx[q].at[1]],
                              sem_s).wait()
        if cnt_sh is not None:
            pltpu.make_async_copy(onesv, cnt_sh.at[idx[q].at[1]],
                                  sem_s).wait()

    load_idx(0, 0)
    load_idx(1, 1)
    load_idx(2, 2)
    wait_idx(0, 0)
    issue_gather(0)
    wait_idx(1, 1)
    issue_gather(1)

    def body(m, carry):
        c0 = 4 * m
        for q in range(4):
            wait_gather(q)

            if q == 0:
                @pl.when(m > 0)
                def _():
                    wait_scatter(3)
            else:
                wait_scatter(q - 1)

            _bf16_to_f32(brows[q % 2], frows, width)
            issue_scatter(q)

            if q == 0:
                load_idx(c0 + 3, 3)
                wait_idx(c0 + 2, 2)
                issue_gather(2)
            elif q == 1:
                @pl.when(m + 1 < niter)
                def _():
                    load_idx(c0 + 4, 0)
                wait_idx(c0 + 3, 3)
                issue_gather(3)
            else:
                @pl.when(m + 1 < niter)
                def _(q=q, c0=c0):
                    load_idx(c0 + q + 3, (q + 3) % 4)
                    wait_idx(c0 + q + 2, (q + 2) % 4)
                    issue_gather((q + 2) % 4)

        return carry

    lax.fori_loop(0, niter, body, 0)

    wait_scatter(3)


_TILE_CHUNKS = [(t * CH, CH) for t in range(ROWS_PER_TILE // CH)]
if ROWS_PER_TILE % CH:
    _TILE_CHUNKS.append((ROWS_PER_TILE - ROWS_PER_TILE % CH,
                         ROWS_PER_TILE % CH))


def _stage_a_body(xc, xp, sd_all, z128, z16, ones_h,
                  aggp, cntp, aggc, cntc,
                  i0, i1, i2, i3, b0, b1, f0,
                  onesv, acc_sh, cnt_sh,
                  sg0, sg1, ss0,
                  si0, si1, si2, si3):
    idx = [i0, i1, i2, i3]
    brows = [b0, b1]
    sem_g = [sg0, sg1]
    sem_i = [si0, si1, si2, si3]

    cid = lax.axis_index("c")
    sid = lax.axis_index("s")
    row0 = sid * ROWS_PER_TILE

    pltpu.sync_copy(z16, onesv)
    pltpu.sync_copy(z128, f0)
    for off, nr in _TILE_CHUNKS:
        pltpu.sync_copy(f0.at[pl.ds(0, nr)],
                        acc_sh.at[pl.ds(row0 + off, nr)])
        pltpu.sync_copy(onesv.at[pl.ds(0, nr)],
                        cnt_sh.at[pl.ds(row0 + off, nr)])
    pltpu.sync_copy(ones_h, onesv)
    plsc.subcore_barrier()

    chunk0 = sid * (M_A * 4)

    @pl.when(cid == 0)
    def _():
        _edge_pipeline(xc, sd_all.at[0], chunk0, M_A, D,
                       idx, brows, f0, sem_g, ss0, sem_i,
                       acc_sh, onesv, cnt_sh)

    @pl.when(cid == 1)
    def _():
        _edge_pipeline(xp, sd_all.at[1], chunk0, M_A, D,
                       idx, brows, f0, sem_g, ss0, sem_i,
                       acc_sh, onesv, cnt_sh)

    plsc.subcore_barrier()

    def _writeback(agg_out, cnt_out):
        for off, nr in _TILE_CHUNKS:
            r = row0 + off
            pltpu.sync_copy(acc_sh.at[pl.ds(r, nr)], f0.at[pl.ds(0, nr)])
            pltpu.sync_copy(f0.at[pl.ds(0, nr)], agg_out.at[pl.ds(r, nr)])
            pltpu.sync_copy(cnt_sh.at[pl.ds(r, nr)], onesv.at[pl.ds(0, nr)])
            pltpu.sync_copy(onesv.at[pl.ds(0, nr)], cnt_out.at[pl.ds(r, nr)])

    @pl.when(cid == 0)
    def _():
        _writeback(aggp, cntp)

    @pl.when(cid == 1)
    def _():
        _writeback(aggc, cntc)


_stage_a = functools.partial(
    pl.kernel,
    out_type=[
        jax.ShapeDtypeStruct((NPAD, D), jnp.float32),
        jax.ShapeDtypeStruct((NPAD, 16), jnp.float32),
        jax.ShapeDtypeStruct((NPAD, D), jnp.float32),
        jax.ShapeDtypeStruct((NPAD, 16), jnp.float32),
    ],
    mesh=_mesh,
    scratch_types=(
        [pltpu.VMEM((2, CH), jnp.int32)] * 4 +
        [pltpu.VMEM((CH, D), jnp.bfloat16)] * 2 +
        [pltpu.VMEM((CH, D), jnp.float32)] +
        [pltpu.VMEM((CH, 16), jnp.float32)] +
        [pltpu.VMEM_SHARED((NPAD, D), jnp.float32),
         pltpu.VMEM_SHARED((NPAD, 16), jnp.float32)]
        + [pltpu.SemaphoreType.DMA] * 7
    ),
    compiler_params=pltpu.CompilerParams(use_tc_tiling_on_sc=False,
                                         needs_layout_passes=False),
)(_stage_a_body)


def _stage_c_body(g, sd_all, z64,
                  agg2,
                  i0, i1, i2, i3, b0, b1, f0,
                  acc_sh,
                  sg0, sg1, ss0,
                  si0, si1, si2, si3):
    idx = [i0, i1, i2, i3]
    brows = [b0, b1]
    sem_g = [sg0, sg1]
    sem_i = [si0, si1, si2, si3]

    cid = lax.axis_index("c")
    sid = lax.axis_index("s")
    row0 = sid * ROWS_PER_TILE

    pltpu.sync_copy(z64, f0)
    for off, nr in _TILE_CHUNKS:
        pltpu.sync_copy(f0.at[pl.ds(0, nr)],
                        acc_sh.at[pl.ds(row0 + off, nr)])
    plsc.subcore_barrier()

    wid = sid * 2 + cid
    chunk0 = wid * (M_C * 4)
    _edge_pipeline(g, sd_all.at[1], chunk0, M_C, DO,
                   idx, brows, f0, sem_g, ss0, sem_i,
                   acc_sh, None, None)

    plsc.subcore_barrier()

    def _writeback(out2d):
        for off, nr in _TILE_CHUNKS:
            r = row0 + off
            pltpu.sync_copy(acc_sh.at[pl.ds(r, nr)], f0.at[pl.ds(0, nr)])
            pltpu.sync_copy(f0.at[pl.ds(0, nr)], out2d.at[pl.ds(r, nr)])

    @pl.when(cid == 0)
    def _():
        _writeback(agg2.at[0])

    @pl.when(cid == 1)
    def _():
        _writeback(agg2.at[1])


_stage_c = functools.partial(
    pl.kernel,
    out_type=[jax.ShapeDtypeStruct((2, NPAD, DO), jnp.float32)],
    mesh=_mesh,
    scratch_types=(
        [pltpu.VMEM((2, CH), jnp.int32)] * 4 +
        [pltpu.VMEM((CH, DO), jnp.bfloat16)] * 2 +
        [pltpu.VMEM((CH, DO), jnp.float32)] +
        [pltpu.VMEM_SHARED((NPAD, DO), jnp.float32)]
        + [pltpu.SemaphoreType.DMA] * 7
    ),
    compiler_params=pltpu.CompilerParams(use_tc_tiling_on_sc=False,
                                         needs_layout_passes=False),
)(_stage_c_body)


_BLK = 1000


def _dot(a, b):
    return jnp.dot(a, b, preferred_element_type=jnp.float32,
                   precision=lax.Precision.HIGHEST)


def _stage_b_kern(aggp, cntp, xp, aggc, cntc, xc,
                  W1cl, b1c, W1cr, W1pl, b1p, W1pr,
                  W2pl, W2pr, WL, WLg, b2p, bL,
                  g_out, z_out):
    mean_p = aggp[...] / jnp.maximum(cntp[:, 0:1], 1.0)
    h_p = jnp.maximum(
        _dot(mean_p, W1cl[...]) + b1c[...] + _dot(xp[...], W1cr[...]), 0.0)
    g_out[...] = _dot(h_p, _dot(W2pl[...], WLg[...])).astype(jnp.bfloat16)

    mean_c = aggc[...] / jnp.maximum(cntc[:, 0:1], 1.0)
    h_c = jnp.maximum(
        _dot(mean_c, W1pl[...]) + b1p[...] + _dot(xc[...], W1pr[...]), 0.0)
    z_out[...] = (_dot(h_c, _dot(W2pr[...], WL[...]))
                  + _dot(b2p[...], WL[...]) + bL[...])


def _stage_b(aggp, cntp, xp, aggc, cntc, xc,
             W1cl, b1c, W1cr, W1pl, b1p, W1pr, W2pl, W2pr, WL, WLg,
             b2p, bL):
    row_spec = lambda w: pl.BlockSpec((_BLK, w), lambda i: (i, 0))
    full = lambda a: pl.BlockSpec(a.shape, lambda i: (0,) * a.ndim)
    return pl.pallas_call(
        _stage_b_kern,
        grid=(N // _BLK,),
        in_specs=[
            row_spec(D), row_spec(16), row_spec(D),
            row_spec(D), row_spec(16), row_spec(D),
            full(W1cl), full(b1c), full(W1cr),
            full(W1pl), full(b1p), full(W1pr),
            full(W2pl), full(W2pr), full(WL), full(WLg), full(b2p),
            full(bL),
        ],
        out_specs=[row_spec(DO), row_spec(DO)],
        out_shape=[
            jax.ShapeDtypeStruct((N, DO), jnp.bfloat16),
            jax.ShapeDtypeStruct((N, DO), jnp.float32),
        ],
    )(aggp, cntp, xp, aggc, cntc, xc,
      W1cl, b1c, W1cr, W1pl, b1p, W1pr, W2pl, W2pr, WL, WLg, b2p, bL)


def _stage_d_kern(p0, p1, cntc, z, out):
    out[...] = ((p0[...] + p1[...]) / jnp.maximum(cntc[:, 0:1], 1.0)
                + z[...])


def _stage_d(p0, p1, cntc, z):
    row_spec = lambda w: pl.BlockSpec((_BLK, w), lambda i: (i, 0))
    return pl.pallas_call(
        _stage_d_kern,
        grid=(N // _BLK,),
        in_specs=[row_spec(DO), row_spec(DO), row_spec(16), row_spec(DO)],
        out_specs=row_spec(DO),
        out_shape=jax.ShapeDtypeStruct((N, DO), jnp.float32),
    )(p0, p1, cntc, z)


def _pad_edges(ei):
    src = ei[0].astype(jnp.int32)
    dst = ei[1].astype(jnp.int32)
    pad = NCHUNK * CH - E
    src = jnp.concatenate([src, jnp.zeros((pad,), jnp.int32)])
    dst = jnp.concatenate([dst, jnp.full((pad,), N, jnp.int32)])
    return jnp.stack([src.reshape(NCHUNK, CH), dst.reshape(NCHUNK, CH)],
                     axis=1)


def kernel(x_customer, x_product, edge_index_cbp, edge_index_pbc,
           W1_cbp_l, b1_cbp, W1_cbp_r, W1_pbc_l, b1_pbc, W1_pbc_r,
           W2_cbp_l, b2_cbp, W2_cbp_r, W2_pbc_l, b2_pbc, W2_pbc_r,
           W_lin, b_lin):
    sd_all = jnp.stack([_pad_edges(edge_index_cbp),
                        _pad_edges(edge_index_pbc)])

    z128 = jnp.zeros((CH, D), jnp.float32)
    z16 = jnp.zeros((CH, 16), jnp.float32)
    z64 = jnp.zeros((CH, DO), jnp.float32)
    ones_h = jnp.ones((CH, 16), jnp.float32)

    xc_bf = x_customer.astype(jnp.bfloat16)
    xp_bf = x_product.astype(jnp.bfloat16)

    aggp, cntp, aggc, cntc = _stage_a(
        xc_bf, xp_bf, sd_all, z128, z16, ones_h)

    perm = jnp.array(_PERM128, jnp.int32)
    g, z = _stage_b(
        aggp[:N], cntp[:N], x_product, aggc[:N], cntc[:N], x_customer,
        W1_cbp_l[perm], b1_cbp.reshape(1, D), W1_cbp_r,
        W1_pbc_l[perm], b1_pbc.reshape(1, D), W1_pbc_r,
        W2_pbc_l, W2_pbc_r, W_lin, W_lin[:, jnp.array(_INV64, jnp.int32)],
        b2_pbc.reshape(1, D), b_lin.reshape(1, DO))

    (agg2,) = _stage_c(g, sd_all, z64)

    return _stage_d(agg2[0, :N], agg2[1, :N], cntc[:N], z)
